# Initial kernel scaffold; baseline (speedup 1.0000x reference)
import jax
import jax.numpy as jnp
from jax.experimental import pallas as pl


def kernel(x, edge_index, W1, b1, W2, b2, bn_gamma, bn_beta, W_skip, b_skip):
    raise NotImplementedError("write your pallas kernel here")


# speedup vs baseline: 7.3086x; 7.3086x over previous
import dataclasses
import functools

import jax
import jax.numpy as jnp
from jax import lax
from jax.experimental import pallas as pl
from jax.experimental.pallas import tpu as pltpu
from jax.experimental.pallas import tpu_sc as plsc

_N = 10000
_D = 256
_E = 160000
_EP = 163840
_RR = _EP // 128
_NP = 10240
_TRASH = 10016
_NC = 2
_NS = 16
_BR = 1000
_NB = _N // _BR

_mesh = plsc.VectorSubcoreMesh(core_axis_name="c", subcore_axis_name="s")

_sc_params = pltpu.CompilerParams()
if "needs_layout_passes" in pltpu.CompilerParams.__dataclass_fields__:
    _sc_params = dataclasses.replace(_sc_params, needs_layout_passes=False)


def _deg_body(dst_hbm, degp_hbm, part, dbuf, stage, red, outbuf):
    c = lax.axis_index("c")
    s = lax.axis_index("s")
    zeros16 = jnp.zeros((16,), jnp.float32)
    ones16 = jnp.ones((16,), jnp.float32)

    @pl.loop(0, _NP, step=16)
    def _(i):
        part[pl.ds(i, 16)] = zeros16

    w = c * _NS + s
    per_w = _EP // (_NC * _NS)
    pltpu.sync_copy(dst_hbm.at[pl.ds(w * per_w, per_w)], dbuf)

    @pl.loop(0, per_w // 16)
    def _(i):
        idx16 = dbuf[pl.ds(i * 16, 16)]
        plsc.addupdate_scatter(part, [idx16], ones16)

    pltpu.sync_copy(part, stage.at[s])
    plsc.subcore_barrier()
    nps = _NP // _NS
    pltpu.sync_copy(stage.at[:, pl.ds(s * nps, nps)], red)

    @pl.loop(0, nps, step=16)
    def _(i):
        acc = red[0, pl.ds(i, 16)]
        for k in range(1, _NS):
            acc = acc + red[k, pl.ds(i, 16)]
        outbuf[pl.ds(i, 16)] = acc

    pltpu.sync_copy(outbuf, degp_hbm.at[pl.ds(c * _NP + s * nps, nps)])


_deg_call = pl.kernel(
    _deg_body,
    out_type=jax.ShapeDtypeStruct((_NC * _NP,), jnp.float32),
    mesh=_mesh,
    scratch_types=[
        pltpu.VMEM((_NP,), jnp.float32),
        pltpu.VMEM((_EP // (_NC * _NS),), jnp.int32),
        pltpu.VMEM_SHARED((_NS, _NP), jnp.float32),
        pltpu.VMEM((_NS, _NP // _NS), jnp.float32),
        pltpu.VMEM((_NP // _NS,), jnp.float32),
    ],
    compiler_params=_sc_params,
)


def _agg_body(h_hbm, src_hbm, dst_hbm, out_hbm, acc, sbuf, dbuf, gbuf):
    c = lax.axis_index("c")
    s = lax.axis_index("s")
    nps = _NP // _NS
    rows_per_s = _RR // _NS

    for k in range(5):
        rows = pl.ds(s * nps + k * 128, 128)
        pltpu.sync_copy(h_hbm.at[c].at[rows], gbuf)
        pltpu.sync_copy(gbuf, acc.at[rows])
    plsc.subcore_barrier()

    pltpu.sync_copy(src_hbm.at[pl.ds(s * rows_per_s, rows_per_s)], sbuf)
    pltpu.sync_copy(dst_hbm.at[pl.ds(s * rows_per_s, rows_per_s)], dbuf)

    @pl.loop(0, rows_per_s)
    def _(r):
        pltpu.sync_copy(h_hbm.at[c].at[sbuf.at[r]], gbuf)
        pltpu.sync_copy(gbuf, acc.at[dbuf.at[r]], add=True)

    plsc.subcore_barrier()
    for k in range(5):
        rows = pl.ds(s * nps + k * 128, 128)
        pltpu.sync_copy(acc.at[rows], gbuf)
        pltpu.sync_copy(gbuf, out_hbm.at[c].at[rows])


_agg_call = pl.kernel(
    _agg_body,
    out_type=jax.ShapeDtypeStruct((_NC, _NP, 128), jnp.float32),
    mesh=_mesh,
    scratch_types=[
        pltpu.VMEM_SHARED((_NP, 128), jnp.float32),
        pltpu.VMEM((_RR // _NS, 128), jnp.int32),
        pltpu.VMEM((_RR // _NS, 128), jnp.int32),
        pltpu.VMEM((128, 128), jnp.float32),
    ],
)


def _dis_body(degp_ref, out_ref):
    deg = degp_ref[0] + degp_ref[1] + 1.0
    out_ref[...] = lax.rsqrt(deg)[:_N, None]


def _dis(degp):
    return pl.pallas_call(
        _dis_body,
        grid=(1,),
        in_specs=[pl.BlockSpec((_NC, _NP), lambda r: (0, 0))],
        out_specs=pl.BlockSpec((_N, 1), lambda r: (0, 0)),
        out_shape=jax.ShapeDtypeStruct((_N, 1), jnp.float32),
    )(degp)


def _mm_scale_body(x_ref, w_ref, dis_ref, out_ref):
    h = jnp.dot(x_ref[...], w_ref[...], preferred_element_type=jnp.float32)
    h = h * dis_ref[...]
    out_ref[0] = h[:, :128]
    out_ref[1] = h[:, 128:]


def _mm_scale(x, w, dis):
    return pl.pallas_call(
        _mm_scale_body,
        grid=(_NB,),
        in_specs=[
            pl.BlockSpec((_BR, _D), lambda r: (r, 0)),
            pl.BlockSpec((_D, _D), lambda r: (0, 0)),
            pl.BlockSpec((_BR, 1), lambda r: (r, 0)),
        ],
        out_specs=pl.BlockSpec((_NC, _BR, 128), lambda r: (0, r, 0)),
        out_shape=jax.ShapeDtypeStruct((_NC, _NP, 128), jnp.float32),
    )(x, w, dis)


def _bn_mm_body(agg_ref, dis_ref, b1_ref, g_ref, be_ref, w2_ref, out_ref,
                stats):
    p = pl.program_id(0)
    r = pl.program_id(1)
    y = jnp.concatenate([agg_ref[0], agg_ref[1]], axis=1) * dis_ref[...]
    y = y + b1_ref[...]

    @pl.when(jnp.logical_and(p == 0, r == 0))
    def _():
        stats[...] = jnp.zeros_like(stats)

    @pl.when(p == 0)
    def _():
        stats[0, :] += jnp.sum(y, axis=0)
        stats[1, :] += jnp.sum(y * y, axis=0)

    @pl.when(p == 1)
    def _():
        mean = stats[0, :] / _N
        var = stats[1, :] / _N - mean * mean
        inv = lax.rsqrt(var + 1e-5)
        yn = g_ref[...] * (y - mean) * inv + be_ref[...]
        h = jnp.maximum(yn, 0.0)
        h2 = jnp.dot(h, w2_ref[...], preferred_element_type=jnp.float32)
        h2 = h2 * dis_ref[...]
        out_ref[0] = h2[:, :128]
        out_ref[1] = h2[:, 128:]


def _bn_mm(agg, dis, b1, g, be, w2):
    return pl.pallas_call(
        _bn_mm_body,
        grid=(2, _NB),
        in_specs=[
            pl.BlockSpec((_NC, _BR, 128), lambda p, r: (0, r, 0)),
            pl.BlockSpec((_BR, 1), lambda p, r: (r, 0)),
            pl.BlockSpec((_D,), lambda p, r: (0,)),
            pl.BlockSpec((_D,), lambda p, r: (0,)),
            pl.BlockSpec((_D,), lambda p, r: (0,)),
            pl.BlockSpec((_D, _D), lambda p, r: (0, 0)),
        ],
        out_specs=pl.BlockSpec((_NC, _BR, 128), lambda p, r: (0, r, 0)),
        out_shape=jax.ShapeDtypeStruct((_NC, _NP, 128), jnp.float32),
        scratch_shapes=[pltpu.VMEM((2, _D), jnp.float32)],
    )(agg, dis, b1, g, be, w2)


def _skip_body(x_ref, w_ref, b_ref, out_ref):
    out_ref[...] = (
        jnp.dot(x_ref[...], w_ref[...], preferred_element_type=jnp.float32)
        + b_ref[...]
    )


def _skip(x, w, b):
    return pl.pallas_call(
        _skip_body,
        grid=(_NB,),
        in_specs=[
            pl.BlockSpec((_BR, _D), lambda r: (r, 0)),
            pl.BlockSpec((_D, _D), lambda r: (0, 0)),
            pl.BlockSpec((_D,), lambda r: (0,)),
        ],
        out_specs=pl.BlockSpec((_BR, _D), lambda r: (r, 0)),
        out_shape=jax.ShapeDtypeStruct((_N, _D), jnp.float32),
    )(x, w, b)


def _final_body(agg_ref, dis_ref, b2_ref, skip_ref, out_ref):
    y = jnp.concatenate([agg_ref[0], agg_ref[1]], axis=1) * dis_ref[...]
    y = jnp.maximum(y + b2_ref[...], 0.0)
    out_ref[...] = jnp.maximum(y + skip_ref[...], 0.0)


def _final(agg, dis, b2, skip):
    return pl.pallas_call(
        _final_body,
        grid=(_NB,),
        in_specs=[
            pl.BlockSpec((_NC, _BR, 128), lambda r: (0, r, 0)),
            pl.BlockSpec((_BR, 1), lambda r: (r, 0)),
            pl.BlockSpec((_D,), lambda r: (0,)),
            pl.BlockSpec((_BR, _D), lambda r: (r, 0)),
        ],
        out_specs=pl.BlockSpec((_BR, _D), lambda r: (r, 0)),
        out_shape=jax.ShapeDtypeStruct((_N, _D), jnp.float32),
    )(agg, dis, b2, skip)


def kernel(x, edge_index, W1, b1, W2, b2, bn_gamma, bn_beta, W_skip, b_skip):
    src = edge_index[0]
    dst = edge_index[1]
    pad = _EP - _E
    src2d = jnp.concatenate(
        [src, jnp.zeros((pad,), jnp.int32)]).reshape(_RR, 128)
    dst2d = jnp.concatenate(
        [dst, jnp.full((pad,), _TRASH, jnp.int32)]).reshape(_RR, 128)

    degp = _deg_call(dst2d.reshape(_EP)).reshape(_NC, _NP)
    dis = _dis(degp)
    h1p = _mm_scale(x, W1, dis)
    agg1 = _agg_call(h1p, src2d, dst2d)
    h2p = _bn_mm(agg1, dis, b1, bn_gamma, bn_beta, W2)
    agg2 = _agg_call(h2p, src2d, dst2d)
    skip = _skip(x, W_skip, b_skip)
    return _final(agg2, dis, b2, skip)


# speedup vs baseline: 7.9823x; 1.0922x over previous
import dataclasses
import functools

import jax
import jax.numpy as jnp
from jax import lax
from jax.experimental import pallas as pl
from jax.experimental.pallas import tpu as pltpu
from jax.experimental.pallas import tpu_sc as plsc

_N = 10000
_D = 256
_E = 160000
_EP = 163840
_RR = _EP // 128
_NP = 10240
_TRASH = 10016
_NC = 2
_NS = 16
_BR = 1000
_NB = _N // _BR

_mesh = plsc.VectorSubcoreMesh(core_axis_name="c", subcore_axis_name="s")

_sc_params = pltpu.CompilerParams()
if "needs_layout_passes" in pltpu.CompilerParams.__dataclass_fields__:
    _sc_params = dataclasses.replace(_sc_params, needs_layout_passes=False)


def _deg_body(dst_hbm, degp_hbm, part, dbuf, stage, red, outbuf):
    c = lax.axis_index("c")
    s = lax.axis_index("s")
    zeros16 = jnp.zeros((16,), jnp.float32)
    ones16 = jnp.ones((16,), jnp.float32)

    @pl.loop(0, _NP, step=16)
    def _(i):
        part[pl.ds(i, 16)] = zeros16

    w = c * _NS + s
    per_w = _EP // (_NC * _NS)
    pltpu.sync_copy(dst_hbm.at[pl.ds(w * per_w, per_w)], dbuf)

    @pl.loop(0, per_w // 16)
    def _(i):
        idx16 = dbuf[pl.ds(i * 16, 16)]
        plsc.addupdate_scatter(part, [idx16], ones16)

    pltpu.sync_copy(part, stage.at[s])
    plsc.subcore_barrier()
    nps = _NP // _NS
    pltpu.sync_copy(stage.at[:, pl.ds(s * nps, nps)], red)

    @pl.loop(0, nps, step=16)
    def _(i):
        acc = red[0, pl.ds(i, 16)]
        for k in range(1, _NS):
            acc = acc + red[k, pl.ds(i, 16)]
        outbuf[pl.ds(i, 16)] = acc

    pltpu.sync_copy(outbuf, degp_hbm.at[pl.ds(c * _NP + s * nps, nps)])


_deg_call = pl.kernel(
    _deg_body,
    out_type=jax.ShapeDtypeStruct((_NC * _NP,), jnp.float32),
    mesh=_mesh,
    scratch_types=[
        pltpu.VMEM((_NP,), jnp.float32),
        pltpu.VMEM((_EP // (_NC * _NS),), jnp.int32),
        pltpu.VMEM_SHARED((_NS, _NP), jnp.float32),
        pltpu.VMEM((_NS, _NP // _NS), jnp.float32),
        pltpu.VMEM((_NP // _NS,), jnp.float32),
    ],
    compiler_params=_sc_params,
)


_K = 2
_IC = 40


def _agg_body(h_hbm, src_hbm, dst_hbm, out_hbm, acc, sbuf, dbuf, gbufs,
              gsem, ssem):
    c = lax.axis_index("c")
    s = lax.axis_index("s")
    nps = _NP // _NS
    rows_per_s = _RR // _NS

    for k in range(5):
        rows = pl.ds(s * nps + k * 128, 128)
        pltpu.sync_copy(h_hbm.at[c].at[rows], gbufs.at[0])
        pltpu.sync_copy(gbufs.at[0], acc.at[rows])
    plsc.subcore_barrier()

    def gather_start(r, k):
        pltpu.async_copy(h_hbm.at[c].at[sbuf.at[r]], gbufs.at[k], gsem.at[k])

    def gather_wait(r, k):
        pltpu.make_async_copy(
            h_hbm.at[c].at[sbuf.at[r]], gbufs.at[k], gsem.at[k]).wait()

    def scat_start(r, k):
        pltpu.async_copy(gbufs.at[k], acc.at[dbuf.at[r]], ssem.at[k],
                         add=True)

    def scat_wait(r, k):
        pltpu.make_async_copy(
            gbufs.at[k], acc.at[dbuf.at[r]], ssem.at[k]).wait()

    for ci in range(rows_per_s // _IC):
        rows = pl.ds(s * rows_per_s + ci * _IC, _IC)
        pltpu.sync_copy(src_hbm.at[rows], sbuf)
        pltpu.sync_copy(dst_hbm.at[rows], dbuf)

        for k in range(_K):
            gather_start(k, k)

        ng = _IC // _K

        @pl.loop(0, ng - 1)
        def _(g):
            base = g * _K
            for k in range(_K):
                gather_wait(base + k, k)
                scat_start(base + k, k)
            for k in range(_K):
                scat_wait(base + k, k)
                gather_start(base + _K + k, k)

        last = (ng - 1) * _K
        for k in range(_K):
            gather_wait(last + k, k)
            scat_start(last + k, k)
        for k in range(_K):
            scat_wait(last + k, k)

    plsc.subcore_barrier()
    for k in range(5):
        rows = pl.ds(s * nps + k * 128, 128)
        pltpu.sync_copy(acc.at[rows], gbufs.at[0])
        pltpu.sync_copy(gbufs.at[0], out_hbm.at[c].at[rows])


_agg_call = pl.kernel(
    _agg_body,
    out_type=jax.ShapeDtypeStruct((_NC, _NP, 128), jnp.float32),
    mesh=_mesh,
    scratch_types=[
        pltpu.VMEM_SHARED((_NP, 128), jnp.float32),
        pltpu.VMEM((_IC, 128), jnp.int32),
        pltpu.VMEM((_IC, 128), jnp.int32),
        pltpu.VMEM((_K, 128, 128), jnp.float32),
        pltpu.SemaphoreType.DMA((_K,)),
        pltpu.SemaphoreType.DMA((_K,)),
    ],
)


def _dis_body(degp_ref, out_ref):
    deg = degp_ref[0] + degp_ref[1] + 1.0
    out_ref[...] = lax.rsqrt(deg)[:_N, None]


def _dis(degp):
    return pl.pallas_call(
        _dis_body,
        grid=(1,),
        in_specs=[pl.BlockSpec((_NC, _NP), lambda r: (0, 0))],
        out_specs=pl.BlockSpec((_N, 1), lambda r: (0, 0)),
        out_shape=jax.ShapeDtypeStruct((_N, 1), jnp.float32),
    )(degp)


def _mm_scale_body(x_ref, w_ref, dis_ref, out_ref):
    h = jnp.dot(x_ref[...], w_ref[...], preferred_element_type=jnp.float32)
    h = h * dis_ref[...]
    out_ref[0] = h[:, :128]
    out_ref[1] = h[:, 128:]


def _mm_scale(x, w, dis):
    return pl.pallas_call(
        _mm_scale_body,
        grid=(_NB,),
        in_specs=[
            pl.BlockSpec((_BR, _D), lambda r: (r, 0)),
            pl.BlockSpec((_D, _D), lambda r: (0, 0)),
            pl.BlockSpec((_BR, 1), lambda r: (r, 0)),
        ],
        out_specs=pl.BlockSpec((_NC, _BR, 128), lambda r: (0, r, 0)),
        out_shape=jax.ShapeDtypeStruct((_NC, _NP, 128), jnp.float32),
    )(x, w, dis)


def _bn_mm_body(agg_ref, dis_ref, b1_ref, g_ref, be_ref, w2_ref, out_ref,
                stats):
    p = pl.program_id(0)
    r = pl.program_id(1)
    y = jnp.concatenate([agg_ref[0], agg_ref[1]], axis=1) * dis_ref[...]
    y = y + b1_ref[...]

    @pl.when(jnp.logical_and(p == 0, r == 0))
    def _():
        stats[...] = jnp.zeros_like(stats)

    @pl.when(p == 0)
    def _():
        stats[0, :] += jnp.sum(y, axis=0)
        stats[1, :] += jnp.sum(y * y, axis=0)

    @pl.when(p == 1)
    def _():
        mean = stats[0, :] / _N
        var = stats[1, :] / _N - mean * mean
        inv = lax.rsqrt(var + 1e-5)
        yn = g_ref[...] * (y - mean) * inv + be_ref[...]
        h = jnp.maximum(yn, 0.0)
        h2 = jnp.dot(h, w2_ref[...], preferred_element_type=jnp.float32)
        h2 = h2 * dis_ref[...]
        out_ref[0] = h2[:, :128]
        out_ref[1] = h2[:, 128:]


def _bn_mm(agg, dis, b1, g, be, w2):
    return pl.pallas_call(
        _bn_mm_body,
        grid=(2, _NB),
        in_specs=[
            pl.BlockSpec((_NC, _BR, 128), lambda p, r: (0, r, 0)),
            pl.BlockSpec((_BR, 1), lambda p, r: (r, 0)),
            pl.BlockSpec((_D,), lambda p, r: (0,)),
            pl.BlockSpec((_D,), lambda p, r: (0,)),
            pl.BlockSpec((_D,), lambda p, r: (0,)),
            pl.BlockSpec((_D, _D), lambda p, r: (0, 0)),
        ],
        out_specs=pl.BlockSpec((_NC, _BR, 128), lambda p, r: (0, r, 0)),
        out_shape=jax.ShapeDtypeStruct((_NC, _NP, 128), jnp.float32),
        scratch_shapes=[pltpu.VMEM((2, _D), jnp.float32)],
    )(agg, dis, b1, g, be, w2)


def _skip_body(x_ref, w_ref, b_ref, out_ref):
    out_ref[...] = (
        jnp.dot(x_ref[...], w_ref[...], preferred_element_type=jnp.float32)
        + b_ref[...]
    )


def _skip(x, w, b):
    return pl.pallas_call(
        _skip_body,
        grid=(_NB,),
        in_specs=[
            pl.BlockSpec((_BR, _D), lambda r: (r, 0)),
            pl.BlockSpec((_D, _D), lambda r: (0, 0)),
            pl.BlockSpec((_D,), lambda r: (0,)),
        ],
        out_specs=pl.BlockSpec((_BR, _D), lambda r: (r, 0)),
        out_shape=jax.ShapeDtypeStruct((_N, _D), jnp.float32),
    )(x, w, b)


def _final_body(agg_ref, dis_ref, b2_ref, skip_ref, out_ref):
    y = jnp.concatenate([agg_ref[0], agg_ref[1]], axis=1) * dis_ref[...]
    y = jnp.maximum(y + b2_ref[...], 0.0)
    out_ref[...] = jnp.maximum(y + skip_ref[...], 0.0)


def _final(agg, dis, b2, skip):
    return pl.pallas_call(
        _final_body,
        grid=(_NB,),
        in_specs=[
            pl.BlockSpec((_NC, _BR, 128), lambda r: (0, r, 0)),
            pl.BlockSpec((_BR, 1), lambda r: (r, 0)),
            pl.BlockSpec((_D,), lambda r: (0,)),
            pl.BlockSpec((_BR, _D), lambda r: (r, 0)),
        ],
        out_specs=pl.BlockSpec((_BR, _D), lambda r: (r, 0)),
        out_shape=jax.ShapeDtypeStruct((_N, _D), jnp.float32),
    )(agg, dis, b2, skip)


def kernel(x, edge_index, W1, b1, W2, b2, bn_gamma, bn_beta, W_skip, b_skip):
    src = edge_index[0]
    dst = edge_index[1]
    pad = _EP - _E
    src2d = jnp.concatenate(
        [src, jnp.zeros((pad,), jnp.int32)]).reshape(_RR, 128)
    dst2d = jnp.concatenate(
        [dst, jnp.full((pad,), _TRASH, jnp.int32)]).reshape(_RR, 128)

    degp = _deg_call(dst2d.reshape(_EP)).reshape(_NC, _NP)
    dis = _dis(degp)
    h1p = _mm_scale(x, W1, dis)
    agg1 = _agg_call(h1p, src2d, dst2d)
    h2p = _bn_mm(agg1, dis, b1, bn_gamma, bn_beta, W2)
    agg2 = _agg_call(h2p, src2d, dst2d)
    skip = _skip(x, W_skip, b_skip)
    return _final(agg2, dis, b2, skip)
